# Initial kernel scaffold; baseline (speedup 1.0000x reference)
import jax
import jax.numpy as jnp
from jax.experimental import pallas as pl


def kernel(x_embed, y_embed):
    raise NotImplementedError("write your pallas kernel here")


# speedup vs baseline: 3.4136x; 3.4136x over previous
import jax
import jax.numpy as jnp
from jax.experimental import pallas as pl

_N = 15000
_D0 = 200
_BQ = 512
_BK = 512
_G = 30
_NP = _G * _BQ
_D = 256

_NEG = -jnp.inf
_BIG = 2 ** 30


def _norm_kernel(x_ref, y_ref, xn_ref, yn_ref):
    for src, dst in ((x_ref, xn_ref), (y_ref, yn_ref)):
        v = src[...]
        n = jnp.sqrt(jnp.sum(v * v, axis=1, keepdims=True))
        dst[...] = v / jnp.maximum(n, 1e-8)


def _merge_top2(v1, i1, v2, i2, cand_v, cand_i):
    gt1 = cand_v > v1
    gt2 = cand_v > v2
    nv2 = jnp.where(gt1, v1, jnp.where(gt2, cand_v, v2))
    ni2 = jnp.where(gt1, i1, jnp.where(gt2, cand_i, i2))
    nv1 = jnp.where(gt1, cand_v, v1)
    ni1 = jnp.where(gt1, cand_i, i1)
    return nv1, ni1, nv2, ni2


def _block_top2_lanes(s, idx):
    m1 = jnp.max(s, axis=1, keepdims=True)
    a1 = jnp.min(jnp.where(s == m1, idx, _BIG), axis=1, keepdims=True)
    s2 = jnp.where(idx == a1, _NEG, s)
    m2 = jnp.max(s2, axis=1, keepdims=True)
    a2 = jnp.min(jnp.where(s2 == m2, idx, _BIG), axis=1, keepdims=True)
    return m1, a1, m2, a2


def _block_top2_sublanes(s, idx):
    m1 = jnp.max(s, axis=0, keepdims=True)
    a1 = jnp.min(jnp.where(s == m1, idx, _BIG), axis=0, keepdims=True)
    s2 = jnp.where(idx == a1, _NEG, s)
    m2 = jnp.max(s2, axis=0, keepdims=True)
    a2 = jnp.min(jnp.where(s2 == m2, idx, _BIG), axis=0, keepdims=True)
    return m1, a1, m2, a2


def _topk_kernel(xn_ref, yn_ref, xv_ref, xi_ref, yv_ref, yi_ref):
    q = pl.program_id(0)
    k = pl.program_id(1)

    @pl.when(jnp.logical_and(q == 0, k == 0))
    def _init():
        xv_ref[...] = jnp.full((2, _NP), _NEG, jnp.float32)
        xi_ref[...] = jnp.zeros((2, _NP), jnp.int32)
        yv_ref[...] = jnp.full((2, _NP), _NEG, jnp.float32)
        yi_ref[...] = jnp.zeros((2, _NP), jnp.int32)

    x = xn_ref[...]
    y = yn_ref[...]
    s = jax.lax.dot_general(
        x, y, (((1,), (1,)), ((), ())),
        preferred_element_type=jnp.float32,
        precision=jax.lax.Precision.DEFAULT)

    col = jax.lax.broadcasted_iota(jnp.int32, (_BQ, _BK), 1)
    row = jax.lax.broadcasted_iota(jnp.int32, (_BQ, _BK), 0)

    sx = jnp.where(k * _BK + col < _N, s, _NEG)
    m1, a1, m2, a2 = _block_top2_lanes(sx, col)
    m1t = jnp.transpose(m1)
    a1t = jnp.transpose(a1) + k * _BK
    m2t = jnp.transpose(m2)
    a2t = jnp.transpose(a2) + k * _BK

    sl = pl.ds(q * _BQ, _BQ)
    v1, i1 = xv_ref[0:1, sl], xi_ref[0:1, sl]
    v2, i2 = xv_ref[1:2, sl], xi_ref[1:2, sl]
    v1, i1, v2, i2 = _merge_top2(v1, i1, v2, i2, m1t, a1t)
    v1, i1, v2, i2 = _merge_top2(v1, i1, v2, i2, m2t, a2t)
    xv_ref[0:1, sl], xi_ref[0:1, sl] = v1, i1
    xv_ref[1:2, sl], xi_ref[1:2, sl] = v2, i2

    sy = jnp.where(q * _BQ + row < _N, s, _NEG)
    c1, b1, c2, b2 = _block_top2_sublanes(sy, row)
    b1 = b1 + q * _BQ
    b2 = b2 + q * _BQ

    sk = pl.ds(k * _BK, _BK)
    w1, j1 = yv_ref[0:1, sk], yi_ref[0:1, sk]
    w2, j2 = yv_ref[1:2, sk], yi_ref[1:2, sk]
    w1, j1, w2, j2 = _merge_top2(w1, j1, w2, j2, c1, b1)
    w1, j1, w2, j2 = _merge_top2(w1, j1, w2, j2, c2, b2)
    yv_ref[0:1, sk], yi_ref[0:1, sk] = w1, j1
    yv_ref[1:2, sk], yi_ref[1:2, sk] = w2, j2


def kernel(x_embed, y_embed):
    xp = jnp.pad(x_embed, ((0, _NP - _N), (0, _D - _D0)))
    yp = jnp.pad(y_embed, ((0, _NP - _N), (0, _D - _D0)))

    nb = 8
    nr = _NP // nb
    xn, yn = pl.pallas_call(
        _norm_kernel,
        grid=(nb,),
        in_specs=[pl.BlockSpec((nr, _D), lambda i: (i, 0)),
                  pl.BlockSpec((nr, _D), lambda i: (i, 0))],
        out_specs=[pl.BlockSpec((nr, _D), lambda i: (i, 0)),
                   pl.BlockSpec((nr, _D), lambda i: (i, 0))],
        out_shape=[jax.ShapeDtypeStruct((_NP, _D), jnp.float32),
                   jax.ShapeDtypeStruct((_NP, _D), jnp.float32)],
    )(xp, yp)

    xv, xi, yv, yi = pl.pallas_call(
        _topk_kernel,
        grid=(_G, _G),
        in_specs=[pl.BlockSpec((_BQ, _D), lambda q, k: (q, 0)),
                  pl.BlockSpec((_BK, _D), lambda q, k: (k, 0))],
        out_specs=[pl.BlockSpec((2, _NP), lambda q, k: (0, 0)),
                   pl.BlockSpec((2, _NP), lambda q, k: (0, 0)),
                   pl.BlockSpec((2, _NP), lambda q, k: (0, 0)),
                   pl.BlockSpec((2, _NP), lambda q, k: (0, 0))],
        out_shape=[jax.ShapeDtypeStruct((2, _NP), jnp.float32),
                   jax.ShapeDtypeStruct((2, _NP), jnp.int32),
                   jax.ShapeDtypeStruct((2, _NP), jnp.float32),
                   jax.ShapeDtypeStruct((2, _NP), jnp.int32)],
    )(xn, yn)

    return (xv[:, :_N].T, xi[:, :_N].T, yv[:, :_N].T, yi[:, :_N].T)
